# Initial kernel scaffold; baseline (speedup 1.0000x reference)
import functools

import jax
import jax.numpy as jnp
from jax import lax
from jax.experimental import pallas as pl
from jax.experimental.pallas import tpu as pltpu
from jax.experimental.pallas import tpu_sc as plsc

N = 10000
E = 320000
D = 128
NT = 8
NG = 16
DQ = 32

NCORE = 2
NSUB = 16
NWORK = NCORE * NSUB

K = 128
CH = 80
E_PAD = NWORK * CH * K

KX = 80
CHX = 4
NPW = CHX * KX
N_PADX = NWORK * NPW

N_ACC = 10016
ZSTRIPE = 626
OSTRIPE = 625

_mesh = plsc.VectorSubcoreMesh(
    core_axis_name="c", subcore_axis_name="s", num_cores=NCORE,
    num_subcores=NSUB)


def _zero_vmem(buf, nrows, ncols):
    per_row = ncols // 16

    def body(i, carry):
        buf[i // per_row, pl.ds((i % per_row) * 16, 16)] = jnp.zeros(
            (16,), jnp.float32)
        return carry

    lax.fori_loop(0, nrows * per_row, body, 0)


def _fill_stripe(zbuf, acc, base):
    for t in range(4):
        pltpu.sync_copy(zbuf, acc.at[pl.ds(base + t * 128, 128)])
    pltpu.sync_copy(zbuf.at[pl.ds(0, ZSTRIPE - 512)],
                    acc.at[pl.ds(base + 512, ZSTRIPE - 512)])


@functools.partial(
    pl.kernel,
    out_type=[
        jax.ShapeDtypeStruct((N_PADX, D), jnp.float32),
        jax.ShapeDtypeStruct((NCORE, N, 16), jnp.float32),
    ],
    mesh=_mesh,
    scratch_types=[
        pltpu.VMEM((CHX, KX), jnp.int32),
        pltpu.VMEM((KX, D), jnp.float32),
        pltpu.VMEM((CH, K), jnp.int32),
        pltpu.VMEM((K, 16), jnp.float32),
        pltpu.VMEM((128, 16), jnp.float32),
        pltpu.VMEM_SHARED((N_ACC, 16), jnp.float32),
        pltpu.SemaphoreType.DMA,
    ],
)
def _sc_prep(ne_hbm, xp_hbm, dstp_hbm, h0_hbm, degp_hbm,
             x_v, rows_v, dst_v, ones_v, zb_v, dacc, sem):
    c = lax.axis_index("c")
    s = lax.axis_index("s")
    wid = c * NSUB + s

    pltpu.sync_copy(xp_hbm.at[wid], x_v)
    for t in range(CHX):
        pltpu.async_copy(ne_hbm.at[x_v.at[t]], rows_v, sem).wait()
        pltpu.sync_copy(rows_v, h0_hbm.at[pl.ds(wid * NPW + t * KX, KX)])

    _zero_vmem(zb_v, 128, 16)
    _fill_stripe(zb_v, dacc, s * ZSTRIPE)

    def fill_ones(i, carry):
        ones_v[i, pl.ds(0, 16)] = jnp.ones((16,), jnp.float32)
        return carry

    lax.fori_loop(0, K, fill_ones, 0)
    pltpu.sync_copy(dstp_hbm.at[wid], dst_v)
    plsc.subcore_barrier()

    def dbody(j, carry):
        pltpu.sync_copy(ones_v, dacc.at[dst_v.at[j]], add=True)
        return carry

    lax.fori_loop(0, CH, dbody, 0)
    plsc.subcore_barrier()
    pltpu.sync_copy(dacc.at[pl.ds(s * OSTRIPE, OSTRIPE)],
                    degp_hbm.at[c, pl.ds(s * OSTRIPE, OSTRIPE)])


@functools.partial(
    pl.kernel,
    out_type=jax.ShapeDtypeStruct((NCORE, N, D), jnp.float32),
    mesh=_mesh,
    scratch_types=[
        pltpu.VMEM((CH, K), jnp.int32),
        pltpu.VMEM((CH, K), jnp.int32),
        pltpu.VMEM((K, D), jnp.float32),
        pltpu.VMEM((K, D), jnp.float32),
        pltpu.VMEM_SHARED((N_ACC, D), jnp.float32),
        pltpu.SemaphoreType.DMA,
        pltpu.SemaphoreType.DMA,
    ],
)
def _sc_agg(h_hbm, srcp_hbm, dstp_hbm, aggp_hbm,
            src_v, dst_v, rows0, rows1, acc, sem0, sem1):
    c = lax.axis_index("c")
    s = lax.axis_index("s")
    wid = c * NSUB + s

    pltpu.sync_copy(srcp_hbm.at[wid], src_v)
    pltpu.sync_copy(dstp_hbm.at[wid], dst_v)

    _zero_vmem(rows0, K, D)
    for t in range(4):
        pltpu.sync_copy(rows0, acc.at[pl.ds(s * ZSTRIPE + t * 128, 128)])
    pltpu.sync_copy(rows0.at[pl.ds(0, ZSTRIPE - 512)],
                    acc.at[pl.ds(s * ZSTRIPE + 512, ZSTRIPE - 512)])
    plsc.subcore_barrier()

    pltpu.async_copy(h_hbm.at[src_v.at[0]], rows0, sem0)
    pltpu.async_copy(h_hbm.at[src_v.at[1]], rows1, sem1)

    def body(jj, carry):
        j0 = jj * 2
        pltpu.make_async_copy(h_hbm.at[src_v.at[0]], rows0, sem0).wait()
        pltpu.sync_copy(rows0, acc.at[dst_v.at[j0]], add=True)
        pltpu.async_copy(h_hbm.at[src_v.at[j0 + 2]], rows0, sem0)
        pltpu.make_async_copy(h_hbm.at[src_v.at[1]], rows1, sem1).wait()
        pltpu.sync_copy(rows1, acc.at[dst_v.at[j0 + 1]], add=True)
        pltpu.async_copy(h_hbm.at[src_v.at[j0 + 3]], rows1, sem1)
        return carry

    lax.fori_loop(0, CH // 2 - 1, body, 0)
    j0 = CH - 2
    pltpu.make_async_copy(h_hbm.at[src_v.at[0]], rows0, sem0).wait()
    pltpu.sync_copy(rows0, acc.at[dst_v.at[j0]], add=True)
    pltpu.make_async_copy(h_hbm.at[src_v.at[1]], rows1, sem1).wait()
    pltpu.sync_copy(rows1, acc.at[dst_v.at[j0 + 1]], add=True)

    plsc.subcore_barrier()
    pltpu.sync_copy(acc.at[pl.ds(s * OSTRIPE, OSTRIPE)],
                    aggp_hbm.at[c, pl.ds(s * OSTRIPE, OSTRIPE)])


def _dotT(a, b):
    return lax.dot_general(a, b, (((1,), (1,)), ((), ())),
                           preferred_element_type=jnp.float32)


def _proj_body(h0_ref, nt_ref, nte_ref, pw_ref, pb_ref, out_ref):
    h0 = h0_ref[0:N, :]
    w1 = pw_ref[:, 0:D]
    w2 = pw_ref[:, D:D + DQ]
    teff = _dotT(nte_ref[...], w2)
    oh = (nt_ref[...] == lax.broadcasted_iota(jnp.int32, (N, NT), 1))
    oh = oh.astype(jnp.float32)
    out = _dotT(h0, w1) + jnp.dot(
        oh, teff, preferred_element_type=jnp.float32) + pb_ref[...]
    out_ref[...] = jnp.maximum(out, 0.0)


def _layer_body(h_ref, aggp_ref, degp_ref, wl_ref, bl_ref, wr_ref,
                g_ref, b_ref, out_ref):
    h = h_ref[...]
    deg = degp_ref[0][:, 0:1] + degp_ref[1][:, 0:1]
    deg = jnp.maximum(deg, 1.0)
    agg = (aggp_ref[0] + aggp_ref[1]) / deg
    hn = _dotT(agg, wl_ref[...]) + bl_ref[...] + _dotT(h, wr_ref[...])
    mu = jnp.mean(hn, axis=-1, keepdims=True)
    var = jnp.mean((hn - mu) ** 2, axis=-1, keepdims=True)
    hn = (hn - mu) / jnp.sqrt(var + 1e-5) * g_ref[...] + b_ref[...]
    out_ref[...] = h + jnp.maximum(hn, 0.0)


def _heads_body(h_ref, batch_ref, gw1_ref, gb1_ref, gw2_ref, gb2_ref,
                gw3_ref, gb3_ref, sw1_ref, sb1_ref, sw2_ref, sb2_ref,
                ge_ref, lg_ref, ss_ref):
    h = h_ref[...]
    oh = (batch_ref[...] == lax.broadcasted_iota(jnp.int32, (N, NG), 1))
    oh = oh.astype(jnp.float32)
    contract0 = (((0,), (0,)), ((), ()))
    gsum = lax.dot_general(oh, h, contract0,
                           preferred_element_type=jnp.float32)
    cnt = lax.dot_general(oh, jnp.ones_like(h), contract0,
                          preferred_element_type=jnp.float32)
    gmean = gsum / jnp.maximum(cnt[:, 0:1], 1.0)
    ge = jnp.concatenate([gmean, gmean], axis=-1)
    z = jnp.maximum(_dotT(ge, gw1_ref[...]) + gb1_ref[...], 0.0)
    z = jnp.maximum(_dotT(z, gw2_ref[...]) + gb2_ref[...], 0.0)
    lg_ref[...] = _dotT(z, gw3_ref[...]) + gb3_ref[...]
    ge_ref[...] = ge
    sc = jnp.maximum(_dotT(h, sw1_ref[...]) + sb1_ref[...], 0.0)
    ss_ref[...] = _dotT(sc, sw2_ref[...]) + sb2_ref[...]


def kernel(params, x, node_type, edge_index, batch):
    p = params
    f32 = jnp.float32

    src = edge_index[0]
    dst = edge_index[1]
    epad = E_PAD - E
    srcp = jnp.concatenate([src, jnp.zeros((epad,), jnp.int32)])
    srcp = srcp.reshape(NWORK, CH, K)
    dstp = jnp.concatenate([dst, jnp.full((epad,), N, jnp.int32)])
    dstp = dstp.reshape(NWORK, CH, K)
    xp = jnp.concatenate([x.astype(jnp.int32),
                          jnp.zeros((N_PADX - N,), jnp.int32)])
    xp = xp.reshape(NWORK, CHX, KX)

    h0_pad, degp = _sc_prep(p['node_embed'], xp, dstp)

    h = pl.pallas_call(
        _proj_body,
        out_shape=jax.ShapeDtypeStruct((N, D), f32),
    )(h0_pad, node_type.reshape(N, 1), p['node_type_embed'],
      p['proj_w'], p['proj_b'].reshape(1, D))

    for lp in p['layers']:
        aggp = _sc_agg(h, srcp, dstp)
        h = pl.pallas_call(
            _layer_body,
            out_shape=jax.ShapeDtypeStruct((N, D), f32),
        )(h, aggp, degp, lp['lin_l_w'], lp['lin_l_b'].reshape(1, D),
          lp['lin_r_w'], lp['ln_g'].reshape(1, D), lp['ln_b'].reshape(1, D))

    gc, sp = p['gc'], p['sp']
    ge, lg, ss = pl.pallas_call(
        _heads_body,
        out_shape=[
            jax.ShapeDtypeStruct((NG, 2 * D), f32),
            jax.ShapeDtypeStruct((NG, 10), f32),
            jax.ShapeDtypeStruct((N, 1), f32),
        ],
    )(h, batch.reshape(N, 1), gc['w1'], gc['b1'].reshape(1, D),
      gc['w2'], gc['b2'].reshape(1, D // 2), gc['w3'],
      gc['b3'].reshape(1, 10), sp['w1'], sp['b1'].reshape(1, D // 2),
      sp['w2'], sp['b2'].reshape(1, 1))

    return lg, ss.reshape(N), h, ge


# speedup vs baseline: 4.8036x; 4.8036x over previous
import functools

import jax
import jax.numpy as jnp
from jax import lax
from jax.experimental import pallas as pl
from jax.experimental.pallas import tpu as pltpu
from jax.experimental.pallas import tpu_sc as plsc

N = 10000
E = 320000
D = 128
DH = D // 2
NT = 8
NG = 16
DQ = 32

NCORE = 2
NSUB = 16
NWORK = NCORE * NSUB

K = 128
CH = 160
E_PAD = NSUB * CH * K

KX = 80
CHX = 4
NPW = CHX * KX
N_PADX = NWORK * NPW

N_ACC = 10240
STRIPE = 640


def _zero_vmem(buf, nrows, ncols):
    per_row = ncols // 16

    def body(i, carry):
        buf[i // per_row, pl.ds((i % per_row) * 16, 16)] = jnp.zeros(
            (16,), jnp.float32)
        return carry

    lax.fori_loop(0, nrows * per_row, body, 0)


def _fill_stripe(zbuf, acc, base):
    for t in range(STRIPE // 128):
        pltpu.sync_copy(zbuf, acc.at[pl.ds(base + t * 128, 128)])


def _sc_prep_body(ne_hbm, xp_hbm, dstp_hbm, h0_hbm, degp_hbm,
                  x_v, rows_v, dst_v, ones_v, zb_v, dacc, sem):
    c = lax.axis_index("c")
    s = lax.axis_index("s")
    wid = c * NSUB + s

    pltpu.sync_copy(xp_hbm.at[wid], x_v)
    for t in range(CHX):
        pltpu.async_copy(ne_hbm.at[x_v.at[t]], rows_v, sem).wait()
        pltpu.sync_copy(rows_v, h0_hbm.at[pl.ds(wid * NPW + t * KX, KX)])

    _zero_vmem(zb_v, 128, 16)
    _fill_stripe(zb_v, dacc, s * STRIPE)

    def fill_ones(i, carry):
        ones_v[i, pl.ds(0, 16)] = jnp.ones((16,), jnp.float32)
        return carry

    lax.fori_loop(0, K, fill_ones, 0)
    pltpu.sync_copy(dstp_hbm.at[s], dst_v)
    plsc.subcore_barrier()

    def dbody(j, carry):
        pltpu.sync_copy(ones_v, dacc.at[dst_v.at[j]], add=True)
        return carry

    lax.fori_loop(c * (CH // 2), (c + 1) * (CH // 2), dbody, 0)
    plsc.subcore_barrier()
    pltpu.sync_copy(dacc.at[pl.ds(s * STRIPE, STRIPE)],
                    degp_hbm.at[c, pl.ds(s * STRIPE, STRIPE)])


def _sc_agg_body(hs_hbm, srcp_hbm, dstp_hbm, agg_hbm,
                 src_v, dst_v, rows0, rows1, acc, sem0, sem1):
    c = lax.axis_index("c")
    s = lax.axis_index("s")

    pltpu.sync_copy(srcp_hbm.at[c, s], src_v)
    pltpu.sync_copy(dstp_hbm.at[s], dst_v)

    _zero_vmem(rows0, K, DH)
    _fill_stripe(rows0, acc, s * STRIPE)
    plsc.subcore_barrier()

    pltpu.async_copy(hs_hbm.at[src_v.at[0]], rows0, sem0)
    pltpu.async_copy(hs_hbm.at[src_v.at[1]], rows1, sem1)

    def body(jj, carry):
        j0 = jj * 2
        pltpu.make_async_copy(hs_hbm.at[src_v.at[0]], rows0, sem0).wait()
        pltpu.sync_copy(rows0, acc.at[dst_v.at[j0]], add=True)
        pltpu.async_copy(hs_hbm.at[src_v.at[j0 + 2]], rows0, sem0)
        pltpu.make_async_copy(hs_hbm.at[src_v.at[1]], rows1, sem1).wait()
        pltpu.sync_copy(rows1, acc.at[dst_v.at[j0 + 1]], add=True)
        pltpu.async_copy(hs_hbm.at[src_v.at[j0 + 3]], rows1, sem1)
        return carry

    lax.fori_loop(0, CH // 2 - 1, body, 0)
    j0 = CH - 2
    pltpu.make_async_copy(hs_hbm.at[src_v.at[0]], rows0, sem0).wait()
    pltpu.sync_copy(rows0, acc.at[dst_v.at[j0]], add=True)
    pltpu.make_async_copy(hs_hbm.at[src_v.at[1]], rows1, sem1).wait()
    pltpu.sync_copy(rows1, acc.at[dst_v.at[j0 + 1]], add=True)

    plsc.subcore_barrier()
    pltpu.sync_copy(acc.at[pl.ds(s * STRIPE, STRIPE)],
                    agg_hbm.at[c, pl.ds(s * STRIPE, STRIPE)])


@functools.lru_cache(maxsize=None)
def _build_sc_kernels():
    mesh = plsc.VectorSubcoreMesh(
        core_axis_name="c", subcore_axis_name="s", num_cores=NCORE,
        num_subcores=NSUB)
    cparams = pltpu.CompilerParams(use_tc_tiling_on_sc=False)
    sc_prep = pl.kernel(
        _sc_prep_body,
        out_type=[
            jax.ShapeDtypeStruct((N_PADX, D), jnp.float32),
            jax.ShapeDtypeStruct((NCORE, N_ACC, 16), jnp.float32),
        ],
        mesh=mesh,
        scratch_types=[
            pltpu.VMEM((CHX, KX), jnp.int32),
            pltpu.VMEM((KX, D), jnp.float32),
            pltpu.VMEM((CH, K), jnp.int32),
            pltpu.VMEM((K, 16), jnp.float32),
            pltpu.VMEM((128, 16), jnp.float32),
            pltpu.VMEM_SHARED((N_ACC, 16), jnp.float32),
            pltpu.SemaphoreType.DMA,
        ],
        compiler_params=cparams,
    )
    sc_agg = pl.kernel(
        _sc_agg_body,
        out_type=jax.ShapeDtypeStruct((NCORE, N_ACC, DH), jnp.float32),
        mesh=mesh,
        scratch_types=[
            pltpu.VMEM((CH, K), jnp.int32),
            pltpu.VMEM((CH, K), jnp.int32),
            pltpu.VMEM((K, DH), jnp.float32),
            pltpu.VMEM((K, DH), jnp.float32),
            pltpu.VMEM_SHARED((N_ACC, DH), jnp.float32),
            pltpu.SemaphoreType.DMA,
            pltpu.SemaphoreType.DMA,
        ],
        compiler_params=cparams,
    )
    return sc_prep, sc_agg


def _dotT(a, b):
    return lax.dot_general(a, b, (((1,), (1,)), ((), ())),
                           preferred_element_type=jnp.float32)


def _split_store(h, hs_ref):
    hs_ref[0:N, :] = h[:, 0:DH]
    hs_ref[N:2 * N, :] = h[:, DH:D]


def _proj_body(h0_ref, nt_ref, nte_ref, pw_ref, pb_ref, out_ref, hs_ref):
    h0 = h0_ref[0:N, :]
    w1 = pw_ref[:, 0:D]
    w2 = pw_ref[:, D:D + DQ]
    teff = _dotT(nte_ref[...], w2)
    oh = (nt_ref[...] == lax.broadcasted_iota(jnp.int32, (N, NT), 1))
    oh = oh.astype(jnp.float32)
    out = _dotT(h0, w1) + jnp.dot(
        oh, teff, preferred_element_type=jnp.float32) + pb_ref[...]
    h = jnp.maximum(out, 0.0)
    out_ref[...] = h
    _split_store(h, hs_ref)


def _layer_body(h_ref, agg_ref, degp_ref, wl_ref, bl_ref, wr_ref,
                g_ref, b_ref, out_ref, hs_ref):
    h = h_ref[...]
    deg = degp_ref[0, 0:N, 0:1] + degp_ref[1, 0:N, 0:1]
    deg = jnp.maximum(deg, 1.0)
    agg = jnp.concatenate([agg_ref[0, 0:N, :], agg_ref[1, 0:N, :]],
                          axis=-1) / deg
    hn = _dotT(agg, wl_ref[...]) + bl_ref[...] + _dotT(h, wr_ref[...])
    mu = jnp.mean(hn, axis=-1, keepdims=True)
    var = jnp.mean((hn - mu) ** 2, axis=-1, keepdims=True)
    hn = (hn - mu) / jnp.sqrt(var + 1e-5) * g_ref[...] + b_ref[...]
    h = h + jnp.maximum(hn, 0.0)
    out_ref[...] = h
    _split_store(h, hs_ref)


def _heads_body(h_ref, batch_ref, gw1_ref, gb1_ref, gw2_ref, gb2_ref,
                gw3_ref, gb3_ref, sw1_ref, sb1_ref, sw2_ref, sb2_ref,
                ge_ref, lg_ref, ss_ref):
    h = h_ref[...]
    oh = (batch_ref[...] == lax.broadcasted_iota(jnp.int32, (N, NG), 1))
    oh = oh.astype(jnp.float32)
    contract0 = (((0,), (0,)), ((), ()))
    gsum = lax.dot_general(oh, h, contract0,
                           preferred_element_type=jnp.float32)
    cnt = lax.dot_general(oh, jnp.ones_like(h), contract0,
                          preferred_element_type=jnp.float32)
    gmean = gsum / jnp.maximum(cnt[:, 0:1], 1.0)
    ge = jnp.concatenate([gmean, gmean], axis=-1)
    z = jnp.maximum(_dotT(ge, gw1_ref[...]) + gb1_ref[...], 0.0)
    z = jnp.maximum(_dotT(z, gw2_ref[...]) + gb2_ref[...], 0.0)
    lg_ref[...] = _dotT(z, gw3_ref[...]) + gb3_ref[...]
    ge_ref[...] = ge
    sc = jnp.maximum(_dotT(h, sw1_ref[...]) + sb1_ref[...], 0.0)
    ss_ref[...] = _dotT(sc, sw2_ref[...]) + sb2_ref[...]


def kernel(params, x, node_type, edge_index, batch):
    p = params
    f32 = jnp.float32

    src = edge_index[0]
    dst = edge_index[1]
    epad = E_PAD - E
    src_flat = jnp.concatenate([src, jnp.zeros((epad,), jnp.int32)])
    srcp = jnp.stack([src_flat, src_flat + N]).reshape(NCORE, NSUB, CH, K)
    dstp = jnp.concatenate([dst, jnp.full((epad,), N, jnp.int32)])
    dstp = dstp.reshape(NSUB, CH, K)
    xp = jnp.concatenate([x.astype(jnp.int32),
                          jnp.zeros((N_PADX - N,), jnp.int32)])
    xp = xp.reshape(NWORK, CHX, KX)

    sc_prep, sc_agg = _build_sc_kernels()

    h0_pad, degp = sc_prep(p['node_embed'], xp, dstp)

    h, hs = pl.pallas_call(
        _proj_body,
        out_shape=[jax.ShapeDtypeStruct((N, D), f32),
                   jax.ShapeDtypeStruct((2 * N, DH), f32)],
    )(h0_pad, node_type.reshape(N, 1), p['node_type_embed'],
      p['proj_w'], p['proj_b'].reshape(1, D))

    for lp in p['layers']:
        aggs = sc_agg(hs, srcp, dstp)
        h, hs = pl.pallas_call(
            _layer_body,
            out_shape=[jax.ShapeDtypeStruct((N, D), f32),
                       jax.ShapeDtypeStruct((2 * N, DH), f32)],
        )(h, aggs, degp, lp['lin_l_w'], lp['lin_l_b'].reshape(1, D),
          lp['lin_r_w'], lp['ln_g'].reshape(1, D), lp['ln_b'].reshape(1, D))

    gc, sp = p['gc'], p['sp']
    ge, lg, ss = pl.pallas_call(
        _heads_body,
        out_shape=[
            jax.ShapeDtypeStruct((NG, 2 * D), f32),
            jax.ShapeDtypeStruct((NG, 10), f32),
            jax.ShapeDtypeStruct((N, 8), f32),
        ],
    )(h, batch.reshape(N, 1), gc['w1'], gc['b1'].reshape(1, D),
      gc['w2'], gc['b2'].reshape(1, D // 2), gc['w3'],
      gc['b3'].reshape(1, 10), sp['w1'], sp['b1'].reshape(1, D // 2),
      jnp.pad(sp['w2'], ((0, 7), (0, 0))),
      jnp.pad(sp['b2'].reshape(1, 1), ((0, 0), (0, 7))))

    return lg, ss[:, 0], h, ge


# speedup vs baseline: 5.8597x; 1.2199x over previous
import functools

import numpy as np

import jax
import jax.numpy as jnp
from jax import lax
from jax.experimental import pallas as pl
from jax.experimental.pallas import tpu as pltpu
from jax.experimental.pallas import tpu_sc as plsc

N = 10000
E = 320000
D = 128
DH = D // 2
NT = 8
NG = 16
DQ = 32

NCORE = 2
NSUB = 16
NWORK = NCORE * NSUB

K = 128
CH = 160
E_PAD = NSUB * CH * K

KX = 80
CHX = 4
NPW = CHX * KX
N_PADX = NWORK * NPW

N_ACC = 10240
STRIPE = 640

_AGG_PERM = np.array([
    64 * (q // 64) + 32 * ((q % 64) // 32)
    + (2 * (q % 32) if (q % 32) < 16 else 2 * ((q % 32) - 16) + 1)
    for q in range(D)])


def _zero_vmem(buf, nrows, ncols):
    per_row = ncols // 16

    def body(i, carry):
        buf[i // per_row, pl.ds((i % per_row) * 16, 16)] = jnp.zeros(
            (16,), jnp.float32)
        return carry

    lax.fori_loop(0, nrows * per_row, body, 0)


def _fill_stripe(zbuf, acc, base):
    for t in range(STRIPE // 128):
        pltpu.sync_copy(zbuf, acc.at[pl.ds(base + t * 128, 128)])


def _sc_prep_body(ne_hbm, xp_hbm, dstp_hbm, h0_hbm, degp_hbm,
                  x_v, rows_v, dst_v, ones_v, zb_v, dacc, sem):
    c = lax.axis_index("c")
    s = lax.axis_index("s")
    wid = c * NSUB + s

    pltpu.sync_copy(xp_hbm.at[wid], x_v)
    for t in range(CHX):
        pltpu.async_copy(ne_hbm.at[x_v.at[t]], rows_v, sem).wait()
        pltpu.sync_copy(rows_v, h0_hbm.at[pl.ds(wid * NPW + t * KX, KX)])

    _zero_vmem(zb_v, 128, 16)
    _fill_stripe(zb_v, dacc, s * STRIPE)

    def fill_ones(i, carry):
        ones_v[i, pl.ds(0, 16)] = jnp.ones((16,), jnp.float32)
        return carry

    lax.fori_loop(0, K, fill_ones, 0)
    pltpu.sync_copy(dstp_hbm.at[s], dst_v)
    plsc.subcore_barrier()

    def dbody(j, carry):
        pltpu.sync_copy(ones_v, dacc.at[dst_v.at[j]], add=True)
        return carry

    lax.fori_loop(c * (CH // 2), (c + 1) * (CH // 2), dbody, 0)
    plsc.subcore_barrier()
    pltpu.sync_copy(dacc.at[pl.ds(s * STRIPE, STRIPE)],
                    degp_hbm.at[c, pl.ds(s * STRIPE, STRIPE)])


def _convert_bf16_rows(bf, rf):
    def crow(i, carry):
        for b in range(DH // 32):
            ev, od = plsc.unpack(bf[i, pl.ds(b * 32, 32)],
                                 format=plsc.PackFormat.INTERLEAVED,
                                 preferred_element_type=jnp.float32)
            rf[i, pl.ds(b * 32, 16)] = ev
            rf[i, pl.ds(b * 32 + 16, 16)] = od
        return carry

    lax.fori_loop(0, K, crow, 0)


def _sc_agg_body(hs_hbm, srcp_hbm, dstp_hbm, agg_hbm,
                 src_v, dst_v, bf0, bf1, rf0, rf1, acc, g0, g1, s0, s1):
    c = lax.axis_index("c")
    s = lax.axis_index("s")

    pltpu.sync_copy(srcp_hbm.at[c, s], src_v)
    pltpu.sync_copy(dstp_hbm.at[s], dst_v)

    _zero_vmem(rf0, K, DH)
    _fill_stripe(rf0, acc, s * STRIPE)
    plsc.subcore_barrier()

    pltpu.async_copy(hs_hbm.at[src_v.at[0]], bf0, g0)
    pltpu.async_copy(hs_hbm.at[src_v.at[1]], bf1, g1)

    pltpu.make_async_copy(hs_hbm.at[src_v.at[0]], bf0, g0).wait()
    _convert_bf16_rows(bf0, rf0)
    pltpu.async_copy(hs_hbm.at[src_v.at[2]], bf0, g0)
    pltpu.async_copy(rf0, acc.at[dst_v.at[0]], s0, add=True)
    pltpu.make_async_copy(hs_hbm.at[src_v.at[1]], bf1, g1).wait()
    _convert_bf16_rows(bf1, rf1)
    pltpu.async_copy(hs_hbm.at[src_v.at[3]], bf1, g1)
    pltpu.async_copy(rf1, acc.at[dst_v.at[1]], s1, add=True)

    def body(jj, carry):
        j0 = jj * 2
        pltpu.make_async_copy(hs_hbm.at[src_v.at[0]], bf0, g0).wait()
        pltpu.make_async_copy(rf0, acc.at[dst_v.at[0]], s0).wait()
        _convert_bf16_rows(bf0, rf0)
        pltpu.async_copy(hs_hbm.at[src_v.at[j0 + 2]], bf0, g0)
        pltpu.async_copy(rf0, acc.at[dst_v.at[j0]], s0, add=True)
        pltpu.make_async_copy(hs_hbm.at[src_v.at[1]], bf1, g1).wait()
        pltpu.make_async_copy(rf1, acc.at[dst_v.at[1]], s1).wait()
        _convert_bf16_rows(bf1, rf1)
        pltpu.async_copy(hs_hbm.at[src_v.at[j0 + 3]], bf1, g1)
        pltpu.async_copy(rf1, acc.at[dst_v.at[j0 + 1]], s1, add=True)
        return carry

    lax.fori_loop(1, CH // 2 - 1, body, 0)

    j0 = CH - 2
    pltpu.make_async_copy(hs_hbm.at[src_v.at[0]], bf0, g0).wait()
    pltpu.make_async_copy(rf0, acc.at[dst_v.at[0]], s0).wait()
    _convert_bf16_rows(bf0, rf0)
    pltpu.async_copy(rf0, acc.at[dst_v.at[j0]], s0, add=True)
    pltpu.make_async_copy(hs_hbm.at[src_v.at[1]], bf1, g1).wait()
    pltpu.make_async_copy(rf1, acc.at[dst_v.at[1]], s1).wait()
    _convert_bf16_rows(bf1, rf1)
    pltpu.async_copy(rf1, acc.at[dst_v.at[j0 + 1]], s1, add=True)
    pltpu.make_async_copy(rf0, acc.at[dst_v.at[0]], s0).wait()
    pltpu.make_async_copy(rf1, acc.at[dst_v.at[1]], s1).wait()

    plsc.subcore_barrier()
    pltpu.sync_copy(acc.at[pl.ds(s * STRIPE, STRIPE)],
                    agg_hbm.at[c, pl.ds(s * STRIPE, STRIPE)])


@functools.lru_cache(maxsize=None)
def _build_sc_kernels():
    mesh = plsc.VectorSubcoreMesh(
        core_axis_name="c", subcore_axis_name="s", num_cores=NCORE,
        num_subcores=NSUB)
    cparams = pltpu.CompilerParams(use_tc_tiling_on_sc=False,
                                   needs_layout_passes=False)
    sc_prep = pl.kernel(
        _sc_prep_body,
        out_type=[
            jax.ShapeDtypeStruct((N_PADX, D), jnp.float32),
            jax.ShapeDtypeStruct((NCORE, N_ACC, 16), jnp.float32),
        ],
        mesh=mesh,
        scratch_types=[
            pltpu.VMEM((CHX, KX), jnp.int32),
            pltpu.VMEM((KX, D), jnp.float32),
            pltpu.VMEM((CH, K), jnp.int32),
            pltpu.VMEM((K, 16), jnp.float32),
            pltpu.VMEM((128, 16), jnp.float32),
            pltpu.VMEM_SHARED((N_ACC, 16), jnp.float32),
            pltpu.SemaphoreType.DMA,
        ],
        compiler_params=cparams,
    )
    sc_agg = pl.kernel(
        _sc_agg_body,
        out_type=jax.ShapeDtypeStruct((NCORE, N_ACC, DH), jnp.float32),
        mesh=mesh,
        scratch_types=[
            pltpu.VMEM((CH, K), jnp.int32),
            pltpu.VMEM((CH, K), jnp.int32),
            pltpu.VMEM((K, DH), jnp.bfloat16),
            pltpu.VMEM((K, DH), jnp.bfloat16),
            pltpu.VMEM((K, DH), jnp.float32),
            pltpu.VMEM((K, DH), jnp.float32),
            pltpu.VMEM_SHARED((N_ACC, DH), jnp.float32),
            pltpu.SemaphoreType.DMA,
            pltpu.SemaphoreType.DMA,
            pltpu.SemaphoreType.DMA,
            pltpu.SemaphoreType.DMA,
        ],
        compiler_params=cparams,
    )
    return sc_prep, sc_agg


def _dotT(a, b):
    return lax.dot_general(a, b, (((1,), (1,)), ((), ())),
                           preferred_element_type=jnp.float32)


def _split_store(h, hs_ref):
    hb = h.astype(jnp.bfloat16)
    hs_ref[0:N, :] = hb[:, 0:DH]
    hs_ref[N:2 * N, :] = hb[:, DH:D]


def _proj_body(h0_ref, nt_ref, nte_ref, pw_ref, pb_ref, out_ref, hs_ref):
    h0 = h0_ref[0:N, :]
    w1 = pw_ref[:, 0:D]
    w2 = pw_ref[:, D:D + DQ]
    teff = _dotT(nte_ref[...], w2)
    oh = (nt_ref[...] == lax.broadcasted_iota(jnp.int32, (N, NT), 1))
    oh = oh.astype(jnp.float32)
    out = _dotT(h0, w1) + jnp.dot(
        oh, teff, preferred_element_type=jnp.float32) + pb_ref[...]
    h = jnp.maximum(out, 0.0)
    out_ref[...] = h
    _split_store(h, hs_ref)


def _layer_body(h_ref, agg_ref, degp_ref, wl_ref, bl_ref, wr_ref,
                g_ref, b_ref, out_ref, hs_ref):
    h = h_ref[...]
    deg = degp_ref[0, 0:N, 0:1] + degp_ref[1, 0:N, 0:1]
    deg = jnp.maximum(deg, 1.0)
    agg = jnp.concatenate([agg_ref[0, 0:N, :], agg_ref[1, 0:N, :]],
                          axis=-1) / deg
    hn = _dotT(agg, wl_ref[...]) + bl_ref[...] + _dotT(h, wr_ref[...])
    mu = jnp.mean(hn, axis=-1, keepdims=True)
    var = jnp.mean((hn - mu) ** 2, axis=-1, keepdims=True)
    hn = (hn - mu) / jnp.sqrt(var + 1e-5) * g_ref[...] + b_ref[...]
    h = h + jnp.maximum(hn, 0.0)
    out_ref[...] = h
    _split_store(h, hs_ref)


def _heads_body(h_ref, batch_ref, gw1_ref, gb1_ref, gw2_ref, gb2_ref,
                gw3_ref, gb3_ref, sw1_ref, sb1_ref, sw2_ref, sb2_ref,
                ge_ref, lg_ref, ss_ref):
    h = h_ref[...]
    oh = (batch_ref[...] == lax.broadcasted_iota(jnp.int32, (N, NG), 1))
    oh = oh.astype(jnp.float32)
    contract0 = (((0,), (0,)), ((), ()))
    gsum = lax.dot_general(oh, h, contract0,
                           preferred_element_type=jnp.float32)
    cnt = lax.dot_general(oh, jnp.ones_like(h), contract0,
                          preferred_element_type=jnp.float32)
    gmean = gsum / jnp.maximum(cnt[:, 0:1], 1.0)
    ge = jnp.concatenate([gmean, gmean], axis=-1)
    z = jnp.maximum(_dotT(ge, gw1_ref[...]) + gb1_ref[...], 0.0)
    z = jnp.maximum(_dotT(z, gw2_ref[...]) + gb2_ref[...], 0.0)
    lg_ref[...] = _dotT(z, gw3_ref[...]) + gb3_ref[...]
    ge_ref[...] = ge
    sc = jnp.maximum(_dotT(h, sw1_ref[...]) + sb1_ref[...], 0.0)
    ss_ref[...] = _dotT(sc, sw2_ref[...]) + sb2_ref[...]


def kernel(params, x, node_type, edge_index, batch):
    p = params
    f32 = jnp.float32

    src = edge_index[0]
    dst = edge_index[1]
    epad = E_PAD - E
    src_flat = jnp.concatenate([src, jnp.zeros((epad,), jnp.int32)])
    srcp = jnp.stack([src_flat, src_flat + N]).reshape(NCORE, NSUB, CH, K)
    dstp = jnp.concatenate([dst, jnp.full((epad,), N, jnp.int32)])
    dstp = dstp.reshape(NSUB, CH, K)
    xp = jnp.concatenate([x.astype(jnp.int32),
                          jnp.zeros((N_PADX - N,), jnp.int32)])
    xp = xp.reshape(NWORK, CHX, KX)

    sc_prep, sc_agg = _build_sc_kernels()

    h0_pad, degp = sc_prep(p['node_embed'], xp, dstp)

    h, hs = pl.pallas_call(
        _proj_body,
        out_shape=[jax.ShapeDtypeStruct((N, D), f32),
                   jax.ShapeDtypeStruct((2 * N, DH), jnp.bfloat16)],
    )(h0_pad, node_type.reshape(N, 1), p['node_type_embed'],
      p['proj_w'], p['proj_b'].reshape(1, D))

    for lp in p['layers']:
        aggs = sc_agg(hs, srcp, dstp)
        h, hs = pl.pallas_call(
            _layer_body,
            out_shape=[jax.ShapeDtypeStruct((N, D), f32),
                       jax.ShapeDtypeStruct((2 * N, DH), jnp.bfloat16)],
        )(h, aggs, degp, lp['lin_l_w'][:, _AGG_PERM],
          lp['lin_l_b'].reshape(1, D),
          lp['lin_r_w'], lp['ln_g'].reshape(1, D), lp['ln_b'].reshape(1, D))

    gc, sp = p['gc'], p['sp']
    ge, lg, ss = pl.pallas_call(
        _heads_body,
        out_shape=[
            jax.ShapeDtypeStruct((NG, 2 * D), f32),
            jax.ShapeDtypeStruct((NG, 10), f32),
            jax.ShapeDtypeStruct((N, 8), f32),
        ],
    )(h, batch.reshape(N, 1), gc['w1'], gc['b1'].reshape(1, D),
      gc['w2'], gc['b2'].reshape(1, D // 2), gc['w3'],
      gc['b3'].reshape(1, 10), sp['w1'], sp['b1'].reshape(1, D // 2),
      jnp.pad(sp['w2'], ((0, 7), (0, 0))),
      jnp.pad(sp['b2'].reshape(1, 1), ((0, 0), (0, 7))))

    return lg, ss[:, 0], h, ge
